# baseline (speedup 1.0000x reference)
import functools

import jax
import jax.numpy as jnp
from jax import lax
from jax.experimental import pallas as pl
from jax.experimental.pallas import tpu as pltpu
from jax.experimental.pallas import tpu_sc as plsc

G0 = 256
G1 = 256
D = 256
NUM_WORKERS = 32
ROWS_PER_WORKER = G0 // NUM_WORKERS
S_ROWS = 2 * G0 - 1
W_ROWS = G1 + ROWS_PER_WORKER - 1


def _stage_body(vals_ref, s_ref):
    rows = jax.lax.broadcasted_iota(jnp.int32, (S_ROWS, 1), 0)
    s_ref[...] = jnp.where(rows < (G0 - 1),
                           vals_ref[0, :][None, :],
                           vals_ref[1, :][None, :])


_stage = pl.pallas_call(
    _stage_body,
    out_shape=jax.ShapeDtypeStruct((S_ROWS, D), jnp.float32),
)


def _sc_body(s_hbm, out_hbm, s_v, sem):
    wid = lax.axis_index("s") * 2 + lax.axis_index("c")
    base = wid * ROWS_PER_WORKER
    pltpu.sync_copy(s_hbm.at[pl.ds(base * D, W_ROWS * D)], s_v)
    copies = [
        pltpu.async_copy(s_v.at[pl.ds(j * D, G1 * D)], out_hbm.at[base + j], sem)
        for j in range(ROWS_PER_WORKER)
    ]
    for c in copies:
        c.wait()


_sc_fill = functools.partial(
    pl.kernel,
    out_type=jax.ShapeDtypeStruct((G0, G1 * D), jnp.float32),
    mesh=plsc.VectorSubcoreMesh(core_axis_name="c", subcore_axis_name="s"),
    scratch_types=[
        pltpu.VMEM((W_ROWS * D,), jnp.float32),
        pltpu.SemaphoreType.DMA,
    ],
)(_sc_body)


def kernel(left_to_right, right_to_left, top_to_bottom, bottom_to_top,
           top_left_to_bottom_right, bottom_right_to_top_left,
           bottom_left_to_top_right, top_right_to_bottom_left):
    vals = jnp.stack([bottom_left_to_top_right, top_right_to_bottom_left])
    s = _stage(vals).reshape(S_ROWS * D)
    out = _sc_fill(s)
    return out.reshape(G0, G1, D)


# speedup vs baseline: 4.1084x; 4.1084x over previous
import jax
import jax.numpy as jnp
from jax.experimental import pallas as pl

G0 = 256
G1 = 256
D = 256
ROWS_PER_BLOCK = 32


def _fill_body(vals_ref, out_ref):
    i = pl.program_id(0)
    rows = jax.lax.broadcasted_iota(jnp.int32, (ROWS_PER_BLOCK, G1, 1), 0)
    cols = jax.lax.broadcasted_iota(jnp.int32, (ROWS_PER_BLOCK, G1, 1), 1)
    pred = (rows + i * ROWS_PER_BLOCK + cols) >= (G0 - 1)
    lo = vals_ref[0, :][None, None, :]
    hi = vals_ref[1, :][None, None, :]
    out_ref[...] = jnp.where(pred, hi, lo)


def kernel(left_to_right, right_to_left, top_to_bottom, bottom_to_top,
           top_left_to_bottom_right, bottom_right_to_top_left,
           bottom_left_to_top_right, top_right_to_bottom_left):
    vals = jnp.stack([bottom_left_to_top_right, top_right_to_bottom_left])
    return pl.pallas_call(
        _fill_body,
        grid=(G0 // ROWS_PER_BLOCK,),
        in_specs=[pl.BlockSpec((2, D), lambda i: (0, 0))],
        out_specs=pl.BlockSpec((ROWS_PER_BLOCK, G1, D), lambda i: (i, 0, 0)),
        out_shape=jax.ShapeDtypeStruct((G0, G1, D), jnp.float32),
    )(vals)
